# baseline (speedup 1.0000x reference)
import functools

import jax
import jax.numpy as jnp
from jax import lax
from jax.experimental import pallas as pl
from jax.experimental.pallas import tpu as pltpu
from jax.experimental.pallas import tpu_sc as plsc

NUM_EMB = 1000000
DIM = 32
QMAX = 255.0

BLK = 8000
TC_GRID = NUM_EMB // BLK
LBLK = BLK // 4
LINES = NUM_EMB // 4


def _minmax_body(w_ref, scale_ref, zp_ref, mn_ref, mx_ref):
    i = pl.program_id(0)

    @pl.when(i == 0)
    def _init():
        mn_ref[0] = 0.0
        mx_ref[0] = 0.0

    w = w_ref[...]
    mn_ref[0] = jnp.minimum(mn_ref[0], jnp.min(w))
    mx_ref[0] = jnp.maximum(mx_ref[0], jnp.max(w))

    @pl.when(i == TC_GRID - 1)
    def _finish():
        mn = mn_ref[0]
        mx = mx_ref[0]
        scale = jnp.maximum((mx - mn) / QMAX, 1e-8)
        zp = jnp.clip(jnp.round(-mn / scale), 0.0, QMAX)
        scale_ref[...] = jnp.full((1, 128), scale, jnp.float32)
        zp_ref[...] = jnp.full((1, 128), zp, jnp.float32)


_minmax = pl.pallas_call(
    _minmax_body,
    grid=(TC_GRID,),
    in_specs=[pl.BlockSpec((BLK, DIM), lambda i: (i, 0))],
    out_specs=[
        pl.BlockSpec((1, 128), lambda i: (0, 0)),
        pl.BlockSpec((1, 128), lambda i: (0, 0)),
    ],
    out_shape=[
        jax.ShapeDtypeStruct((1, 128), jnp.float32),
        jax.ShapeDtypeStruct((1, 128), jnp.float32),
    ],
    scratch_shapes=[
        pltpu.SMEM((1,), jnp.float32),
        pltpu.SMEM((1,), jnp.float32),
    ],
)


def _transform_body(w_ref, m_ref, s_ref, z_ref, o_ref):
    s = s_ref[...]
    zp = z_ref[...]
    w = w_ref[...]
    m = m_ref[...]
    sd = s[:, :DIM]
    zd = zp[:, :DIM]
    t = w / sd + zd
    q = jnp.clip(jnp.round(t), 0.0, QMAX)
    wq = (q - zd) * sd
    noise = jnp.where(m, 0.0, wq - w)
    wt = jnp.clip(w, -sd * zd, sd * (QMAX - zd)) + noise
    for qq in range(4):
        o_ref[:, pl.ds(qq * DIM, DIM)] = wt[qq * LBLK:(qq + 1) * LBLK, :]


_transform = pl.pallas_call(
    _transform_body,
    grid=(TC_GRID,),
    in_specs=[
        pl.BlockSpec((BLK, DIM), lambda i: (i, 0)),
        pl.BlockSpec((BLK, DIM), lambda i: (i, 0)),
        pl.BlockSpec((1, 128), lambda i: (0, 0)),
        pl.BlockSpec((1, 128), lambda i: (0, 0)),
    ],
    out_specs=pl.BlockSpec((LBLK, 128), lambda i: (i, 0)),
    out_shape=jax.ShapeDtypeStruct((LINES, 128), jnp.float32),
)

B_TOTAL = 4096 * 50
NUM_WORKERS = 32
B_PER_W = B_TOTAL // NUM_WORKERS
CHUNK = 640
NCHUNK = B_PER_W // CHUNK
SUB = 128
NSUB = CHUNK // SUB

_sc_mesh = plsc.VectorSubcoreMesh(core_axis_name="c", subcore_axis_name="s")


@functools.partial(
    pl.kernel,
    mesh=_sc_mesh,
    out_type=jax.ShapeDtypeStruct((B_TOTAL // 4, 128), jnp.float32),
    scratch_types=[
        pltpu.VMEM((CHUNK,), jnp.int32),
        pltpu.VMEM((CHUNK,), jnp.int32),
        pltpu.VMEM((CHUNK,), jnp.int32),
        pltpu.VMEM((CHUNK, 128), jnp.float32),
        pltpu.VMEM((CHUNK // 4, 128), jnp.float32),
        pltpu.SemaphoreType.DMA,
    ],
    compiler_params=pltpu.CompilerParams(needs_layout_passes=False),
)
def _sc_gather(idx_hbm, tab_hbm, out_hbm,
               idx_v, line_v, qcol_v, g_v, o_v, sem):
    wid = lax.axis_index("s") * 2 + lax.axis_index("c")
    base = wid * B_PER_W

    def do_chunk(c, carry):
        off = pl.multiple_of(base + c * CHUNK, CHUNK)
        pltpu.sync_copy(idx_hbm.at[pl.ds(off, CHUNK)], idx_v)

        def decode(v, carry2):
            iv = idx_v[pl.ds(v * 16, 16)]
            blk = iv // BLK
            rem = iv - blk * BLK
            qq = rem // LBLK
            line_v[pl.ds(v * 16, 16)] = blk * LBLK + (rem - qq * LBLK)
            qcol_v[pl.ds(v * 16, 16)] = qq * DIM
            return carry2

        lax.fori_loop(0, CHUNK // 16, decode, 0)
        cps = []
        for sub in range(NSUB):
            cps.append(pltpu.async_copy(
                tab_hbm.at[line_v.at[pl.ds(sub * SUB, SUB)]],
                g_v.at[pl.ds(sub * SUB, SUB)], sem))
        for cp in cps:
            cp.wait()

        def do_group(g, carry2):
            qv = qcol_v[pl.ds(g * 16, 16)]
            for k in range(16):
                b = g * 16 + k
                orow = g * 4 + (k >> 2)
                for j in range(2):
                    o_v[orow, pl.ds((k & 3) * DIM + j * 16, 16)] = (
                        g_v[b, pl.ds(qv[k] + j * 16, 16)])
            return carry2

        lax.fori_loop(0, CHUNK // 16, do_group, 0)
        pltpu.sync_copy(
            o_v,
            out_hbm.at[pl.ds(pl.multiple_of(off // 4, CHUNK // 4), CHUNK // 4)])
        return carry

    lax.fori_loop(0, NCHUNK, do_chunk, 0)


def kernel(input, weight, mask):
    scale_r, zp_r = _minmax(weight)
    table = _transform(weight, mask, scale_r, zp_r)
    idx = input.reshape(-1)
    out4 = _sc_gather(idx, table)
    return out4.reshape(input.shape + (DIM,))


# speedup vs baseline: 1.1533x; 1.1533x over previous
import functools

import jax
import jax.numpy as jnp
from jax import lax
from jax.experimental import pallas as pl
from jax.experimental.pallas import tpu as pltpu
from jax.experimental.pallas import tpu_sc as plsc

NUM_EMB = 1000000
DIM = 32
QMAX = 255.0

BLK = 8000
TC_GRID = NUM_EMB // BLK
LBLK = BLK // 4
LINES = NUM_EMB // 4


def _minmax_body(w_ref, scale_ref, zp_ref, mn_ref, mx_ref):
    i = pl.program_id(0)

    @pl.when(i == 0)
    def _init():
        mn_ref[0] = 0.0
        mx_ref[0] = 0.0

    w = w_ref[...]
    mn_ref[0] = jnp.minimum(mn_ref[0], jnp.min(w))
    mx_ref[0] = jnp.maximum(mx_ref[0], jnp.max(w))

    @pl.when(i == TC_GRID - 1)
    def _finish():
        mn = mn_ref[0]
        mx = mx_ref[0]
        scale = jnp.maximum((mx - mn) / QMAX, 1e-8)
        zp = jnp.clip(jnp.round(-mn / scale), 0.0, QMAX)
        scale_ref[...] = jnp.full((1, 128), scale, jnp.float32)
        zp_ref[...] = jnp.full((1, 128), zp, jnp.float32)


_minmax = pl.pallas_call(
    _minmax_body,
    grid=(TC_GRID,),
    in_specs=[pl.BlockSpec((LBLK, 128), lambda i: (i, 0))],
    out_specs=[
        pl.BlockSpec((1, 128), lambda i: (0, 0)),
        pl.BlockSpec((1, 128), lambda i: (0, 0)),
    ],
    out_shape=[
        jax.ShapeDtypeStruct((1, 128), jnp.float32),
        jax.ShapeDtypeStruct((1, 128), jnp.float32),
    ],
    scratch_shapes=[
        pltpu.SMEM((1,), jnp.float32),
        pltpu.SMEM((1,), jnp.float32),
    ],
)


def _transform_body(w_ref, m_ref, s_ref, z_ref, o_ref):
    s = s_ref[...]
    zp = z_ref[...]
    w = w_ref[...]
    m = m_ref[...]
    t = w / s + zp
    q = jnp.clip(jnp.round(t), 0.0, QMAX)
    wq = (q - zp) * s
    noise = jnp.where(m != 0, 0.0, wq - w)
    o_ref[...] = jnp.clip(w, -s * zp, s * (QMAX - zp)) + noise


_transform = pl.pallas_call(
    _transform_body,
    grid=(TC_GRID,),
    in_specs=[
        pl.BlockSpec((LBLK, 128), lambda i: (i, 0)),
        pl.BlockSpec((LBLK, 128), lambda i: (i, 0)),
        pl.BlockSpec((1, 128), lambda i: (0, 0)),
        pl.BlockSpec((1, 128), lambda i: (0, 0)),
    ],
    out_specs=pl.BlockSpec((LBLK, 128), lambda i: (i, 0)),
    out_shape=jax.ShapeDtypeStruct((LINES, 128), jnp.float32),
)

B_TOTAL = 4096 * 50
NUM_WORKERS = 32
B_PER_W = B_TOTAL // NUM_WORKERS
CHUNK = 640
NCHUNK = B_PER_W // CHUNK
SUB = 128
NSUB = CHUNK // SUB

_sc_mesh = plsc.VectorSubcoreMesh(core_axis_name="c", subcore_axis_name="s")


@functools.partial(
    pl.kernel,
    mesh=_sc_mesh,
    out_type=jax.ShapeDtypeStruct((B_TOTAL // 4, 128), jnp.float32),
    scratch_types=[
        pltpu.VMEM((CHUNK,), jnp.int32),
        pltpu.VMEM((CHUNK,), jnp.int32),
        pltpu.VMEM((CHUNK,), jnp.int32),
        pltpu.VMEM((CHUNK, 128), jnp.float32),
        pltpu.VMEM((CHUNK // 4, 128), jnp.float32),
        pltpu.SemaphoreType.DMA,
    ],
    compiler_params=pltpu.CompilerParams(needs_layout_passes=False),
)
def _sc_gather(idx_hbm, tab_hbm, out_hbm,
               idx_v, line_v, qcol_v, g_v, o_v, sem):
    wid = lax.axis_index("s") * 2 + lax.axis_index("c")
    base = wid * B_PER_W

    def do_chunk(c, carry):
        off = pl.multiple_of(base + c * CHUNK, CHUNK)
        pltpu.sync_copy(idx_hbm.at[pl.ds(off, CHUNK)], idx_v)

        def decode(v, carry2):
            iv = idx_v[pl.ds(v * 16, 16)]
            line_v[pl.ds(v * 16, 16)] = iv >> 2
            qcol_v[pl.ds(v * 16, 16)] = (iv & 3) * DIM
            return carry2

        lax.fori_loop(0, CHUNK // 16, decode, 0)
        cps = []
        for sub in range(NSUB):
            cps.append(pltpu.async_copy(
                tab_hbm.at[line_v.at[pl.ds(sub * SUB, SUB)]],
                g_v.at[pl.ds(sub * SUB, SUB)], sem))
        for cp in cps:
            cp.wait()

        def do_group(g, carry2):
            qv = qcol_v[pl.ds(g * 16, 16)]
            for k in range(16):
                b = g * 16 + k
                orow = g * 4 + (k >> 2)
                for j in range(2):
                    o_v[orow, pl.ds((k & 3) * DIM + j * 16, 16)] = (
                        g_v[b, pl.ds(qv[k] + j * 16, 16)])
            return carry2

        lax.fori_loop(0, CHUNK // 16, do_group, 0)
        pltpu.sync_copy(
            o_v,
            out_hbm.at[pl.ds(pl.multiple_of(off // 4, CHUNK // 4), CHUNK // 4)])
        return carry

    lax.fori_loop(0, NCHUNK, do_chunk, 0)


def kernel(input, weight, mask):
    w4 = weight.reshape(LINES, 128)
    m4 = mask.astype(jnp.uint8).reshape(LINES, 128)
    scale_r, zp_r = _minmax(w4)
    table = _transform(w4, m4, scale_r, zp_r)
    idx = input.reshape(-1)
    out4 = _sc_gather(idx, table)
    return out4.reshape(input.shape + (DIM,))


# speedup vs baseline: 1.1542x; 1.0008x over previous
import functools

import jax
import jax.numpy as jnp
from jax import lax
from jax.experimental import pallas as pl
from jax.experimental.pallas import tpu as pltpu
from jax.experimental.pallas import tpu_sc as plsc

NUM_EMB = 1000000
DIM = 32
QMAX = 255.0

BLK = 8000
TC_GRID = NUM_EMB // BLK
LBLK = BLK // 4
LINES = NUM_EMB // 4


def _minmax_body(w_ref, scale_ref, zp_ref, mn_ref, mx_ref):
    i = pl.program_id(0)

    @pl.when(i == 0)
    def _init():
        mn_ref[0] = 0.0
        mx_ref[0] = 0.0

    w = w_ref[...]
    mn_ref[0] = jnp.minimum(mn_ref[0], jnp.min(w))
    mx_ref[0] = jnp.maximum(mx_ref[0], jnp.max(w))

    @pl.when(i == TC_GRID - 1)
    def _finish():
        mn = mn_ref[0]
        mx = mx_ref[0]
        scale = jnp.maximum((mx - mn) / QMAX, 1e-8)
        zp = jnp.clip(jnp.round(-mn / scale), 0.0, QMAX)
        scale_ref[...] = jnp.full((1, 128), scale, jnp.float32)
        zp_ref[...] = jnp.full((1, 128), zp, jnp.float32)


_minmax = pl.pallas_call(
    _minmax_body,
    grid=(TC_GRID,),
    in_specs=[pl.BlockSpec((LBLK, 128), lambda i: (i, 0))],
    out_specs=[
        pl.BlockSpec((1, 128), lambda i: (0, 0)),
        pl.BlockSpec((1, 128), lambda i: (0, 0)),
    ],
    out_shape=[
        jax.ShapeDtypeStruct((1, 128), jnp.float32),
        jax.ShapeDtypeStruct((1, 128), jnp.float32),
    ],
    scratch_shapes=[
        pltpu.SMEM((1,), jnp.float32),
        pltpu.SMEM((1,), jnp.float32),
    ],
    compiler_params=pltpu.CompilerParams(allow_input_fusion=[True]),
)


def _transform_body(w_ref, m_ref, s_ref, z_ref, o_ref):
    s = s_ref[...]
    zp = z_ref[...]
    w = w_ref[...]
    m = m_ref[...]
    t = w / s + zp
    q = jnp.clip(jnp.round(t), 0.0, QMAX)
    wq = (q - zp) * s
    noise = jnp.where(m != 0, 0.0, wq - w)
    o_ref[...] = jnp.clip(w, -s * zp, s * (QMAX - zp)) + noise


_transform = pl.pallas_call(
    _transform_body,
    grid=(TC_GRID,),
    in_specs=[
        pl.BlockSpec((LBLK, 128), lambda i: (i, 0)),
        pl.BlockSpec((LBLK, 128), lambda i: (i, 0)),
        pl.BlockSpec((1, 128), lambda i: (0, 0)),
        pl.BlockSpec((1, 128), lambda i: (0, 0)),
    ],
    out_specs=pl.BlockSpec((LBLK, 128), lambda i: (i, 0)),
    out_shape=jax.ShapeDtypeStruct((LINES, 128), jnp.float32),
    compiler_params=pltpu.CompilerParams(
        allow_input_fusion=[True, True, False, False]),
)

B_TOTAL = 4096 * 50
NUM_WORKERS = 32
B_PER_W = B_TOTAL // NUM_WORKERS
CHUNK = 640
NCHUNK = B_PER_W // CHUNK
SUB = 128
NSUB = CHUNK // SUB

_sc_mesh = plsc.VectorSubcoreMesh(core_axis_name="c", subcore_axis_name="s")


@functools.partial(
    pl.kernel,
    mesh=_sc_mesh,
    out_type=jax.ShapeDtypeStruct((B_TOTAL // 4, 128), jnp.float32),
    scratch_types=[
        pltpu.VMEM((CHUNK,), jnp.int32),
        pltpu.VMEM((CHUNK,), jnp.int32),
        pltpu.VMEM((CHUNK,), jnp.int32),
        pltpu.VMEM((CHUNK, 128), jnp.float32),
        pltpu.VMEM((CHUNK // 4, 128), jnp.float32),
        pltpu.SemaphoreType.DMA,
    ],
    compiler_params=pltpu.CompilerParams(needs_layout_passes=False),
)
def _sc_gather(idx_hbm, tab_hbm, out_hbm,
               idx_v, line_v, qcol_v, g_v, o_v, sem):
    wid = lax.axis_index("s") * 2 + lax.axis_index("c")
    base = wid * B_PER_W

    def do_chunk(c, carry):
        off = pl.multiple_of(base + c * CHUNK, CHUNK)
        pltpu.sync_copy(idx_hbm.at[pl.ds(off, CHUNK)], idx_v)

        def decode(v, carry2):
            iv = idx_v[pl.ds(v * 16, 16)]
            line_v[pl.ds(v * 16, 16)] = iv >> 2
            qcol_v[pl.ds(v * 16, 16)] = (iv & 3) * DIM
            return carry2

        lax.fori_loop(0, CHUNK // 16, decode, 0)
        cps = []
        for sub in range(NSUB):
            cps.append(pltpu.async_copy(
                tab_hbm.at[line_v.at[pl.ds(sub * SUB, SUB)]],
                g_v.at[pl.ds(sub * SUB, SUB)], sem))
        for cp in cps:
            cp.wait()

        def do_group(g, carry2):
            qv = qcol_v[pl.ds(g * 16, 16)]
            for k in range(16):
                b = g * 16 + k
                orow = g * 4 + (k >> 2)
                for j in range(2):
                    o_v[orow, pl.ds((k & 3) * DIM + j * 16, 16)] = (
                        g_v[b, pl.ds(qv[k] + j * 16, 16)])
            return carry2

        lax.fori_loop(0, CHUNK // 16, do_group, 0)
        pltpu.sync_copy(
            o_v,
            out_hbm.at[pl.ds(pl.multiple_of(off // 4, CHUNK // 4), CHUNK // 4)])
        return carry

    lax.fori_loop(0, NCHUNK, do_chunk, 0)


def kernel(input, weight, mask):
    w4 = weight.reshape(LINES, 128)
    m4 = mask.astype(jnp.uint8).reshape(LINES, 128)
    scale_r, zp_r = _minmax(w4)
    table = _transform(w4, m4, scale_r, zp_r)
    idx = input.reshape(-1)
    out4 = _sc_gather(idx, table)
    return out4.reshape(input.shape + (DIM,))
